# baseline (speedup 1.0000x reference)
import functools
import math

import jax
import jax.numpy as jnp
from jax import lax
from jax.experimental import pallas as pl
from jax.experimental.pallas import tpu as pltpu
from jax.experimental.pallas import tpu_sc as plsc

D_MODEL = 64
SCALE = math.sqrt(D_MODEL)
LANES = 16
CHUNK = 128


@functools.partial(jax.jit, static_argnums=(2,))
def _embed(x_flat, table, b_flat):
    info = plsc.get_sparse_core_info()
    nc, ns = info.num_cores, info.num_subcores
    nw = nc * ns
    b_per_w = b_flat // nw
    n_chunks = b_per_w // CHUNK
    mesh = plsc.VectorSubcoreMesh(core_axis_name="c", subcore_axis_name="s")

    @functools.partial(
        pl.kernel,
        mesh=mesh,
        out_type=jax.ShapeDtypeStruct((b_flat, D_MODEL), jnp.float32),
        scratch_types=[
            pltpu.VMEM((b_per_w,), jnp.int32),
            pltpu.VMEM((CHUNK, D_MODEL), jnp.float32),
            pltpu.SemaphoreType.DMA,
        ],
        compiler_params=pltpu.CompilerParams(use_tc_tiling_on_sc=False),
    )
    def k(x_hbm, table_hbm, out_hbm, idx_v, rows_v, sem):
        wid = lax.axis_index("s") * nc + lax.axis_index("c")
        base = wid * b_per_w
        pltpu.sync_copy(x_hbm.at[pl.ds(base, b_per_w)], idx_v)

        def chunk_body(j, carry):
            pltpu.async_copy(
                table_hbm.at[idx_v.at[pl.ds(j * CHUNK, CHUNK)]], rows_v, sem
            ).wait()

            def row_body(i, c):
                for k2 in range(D_MODEL // LANES):
                    sl = (i, pl.ds(k2 * LANES, LANES))
                    rows_v[sl] = rows_v[sl] * SCALE
                return c

            lax.fori_loop(0, CHUNK, row_body, 0)
            pltpu.sync_copy(rows_v, out_hbm.at[pl.ds(base + j * CHUNK, CHUNK)])
            return carry

        lax.fori_loop(0, n_chunks, chunk_body, 0)

    return k(x_flat, table)


def kernel(x, embedding_weight):
    b, s = x.shape
    x_flat = x.reshape(b * s).astype(jnp.int32)
    out = _embed(x_flat, embedding_weight, b * s)
    return out.reshape(b, s, D_MODEL)


# speedup vs baseline: 1.2114x; 1.2114x over previous
import functools
import math

import jax
import jax.numpy as jnp
from jax import lax
from jax.experimental import pallas as pl
from jax.experimental.pallas import tpu as pltpu
from jax.experimental.pallas import tpu_sc as plsc

D_MODEL = 64
SCALE = math.sqrt(D_MODEL)
LANES = 16
CHUNK = 128
NBUF = 8
LEAD = 4


@functools.partial(jax.jit, static_argnums=(2,))
def _embed(x_flat, table, b_flat):
    info = plsc.get_sparse_core_info()
    nc, ns = info.num_cores, info.num_subcores
    nw = nc * ns
    b_per_w = b_flat // nw
    n_chunks = b_per_w // CHUNK
    n_groups = n_chunks // NBUF
    mesh = plsc.VectorSubcoreMesh(core_axis_name="c", subcore_axis_name="s")

    @functools.partial(
        pl.kernel,
        mesh=mesh,
        out_type=jax.ShapeDtypeStruct((b_flat, D_MODEL), jnp.float32),
        scratch_types=[
            pltpu.VMEM((b_per_w,), jnp.int32),
            pltpu.VMEM((NBUF, CHUNK, D_MODEL), jnp.float32),
            pltpu.SemaphoreType.DMA((NBUF,)),
        ],
        compiler_params=pltpu.CompilerParams(use_tc_tiling_on_sc=False),
    )
    def k(x_hbm, table_hbm, out_hbm, idx_v, rows_v, sems):
        wid = lax.axis_index("s") * nc + lax.axis_index("c")
        base = wid * b_per_w
        pltpu.sync_copy(x_hbm.at[pl.ds(base, b_per_w)], idx_v)

        def gather_start(j, b):
            pltpu.async_copy(
                table_hbm.at[idx_v.at[pl.ds(j * CHUNK, CHUNK)]],
                rows_v.at[b],
                sems.at[b],
            )

        def gather_wait(b):
            pltpu.make_async_copy(
                table_hbm.at[idx_v.at[pl.ds(0, CHUNK)]], rows_v.at[b], sems.at[b]
            ).wait()

        def store_start(j, b):
            pltpu.async_copy(
                rows_v.at[b], out_hbm.at[pl.ds(base + j * CHUNK, CHUNK)], sems.at[b]
            )

        def store_wait(b):
            pltpu.make_async_copy(
                rows_v.at[b], out_hbm.at[pl.ds(0, CHUNK)], sems.at[b]
            ).wait()

        for b in range(LEAD):
            gather_start(b, b)

        def group(g, carry):
            for b in range(NBUF):
                j = g * NBUF + b
                bl = (b + LEAD) % NBUF
                jl = j + LEAD

                @pl.when(jl < n_chunks)
                def _():
                    @pl.when(jl >= NBUF)
                    def _():
                        store_wait(bl)

                    gather_start(jl, bl)

                gather_wait(b)

                @plsc.parallel_loop(0, CHUNK, step=1, unroll=4)
                def _(i):
                    for k2 in range(D_MODEL // LANES):
                        sl = (b, i, pl.ds(k2 * LANES, LANES))
                        rows_v[sl] = rows_v[sl] * SCALE

                store_start(j, b)
            return carry

        lax.fori_loop(0, n_groups, group, 0)
        for b in range(NBUF - LEAD, NBUF):
            store_wait(b)
        for b in range(0, NBUF - LEAD):
            store_wait(b)

    return k(x_flat, table)


def kernel(x, embedding_weight):
    b, s = x.shape
    x_flat = x.reshape(b * s).astype(jnp.int32)
    out = _embed(x_flat, embedding_weight, b * s)
    return out.reshape(b, s, D_MODEL)
